# baseline (speedup 1.0000x reference)
import jax
import jax.numpy as jnp
from jax.experimental import pallas as pl
from jax.experimental.pallas import tpu as pltpu

_B = 16384
_LD = 2048
_ED = 1024
_HID = 1024
_K = 32
_BLK = 256
_NBLK = _B // _BLK
_BLK2 = 512
_NBLK2 = _B // _BLK2
_DELTA_BASE = 0.01
_VOL_T = 0.5


def _bf16_round(x):
    u = jax.lax.bitcast_convert_type(x, jnp.int32)
    u = u + jnp.int32(0x7FFF) + ((u >> 16) & jnp.int32(1))
    u = u & jnp.int32(-65536)
    return jax.lax.bitcast_convert_type(u, jnp.float32)


def _f2u(xf):
    b = jax.lax.bitcast_convert_type(xf, jnp.int32)
    return b ^ ((b >> 31) & jnp.int32(0x7FFFFFFF))


def _u2f(u):
    u2 = u ^ ((u >> 31) & jnp.int32(0x7FFFFFFF))
    return jax.lax.bitcast_convert_type(u2, jnp.float32)


_KF = float(_K)


def _vol_body(S_ref, E_ref, W1_ref, b1_ref, W2_ref, b2_ref, vol_ref, musum_ref):
    i = pl.program_id(0)

    x = S_ref[...]
    row_sum = jnp.sum(x, axis=1)
    ones = jnp.ones((_LD, 128), jnp.float32)

    def cnt_ge(pf):
        ind = jnp.where(x >= pf[:, None], 1.0, 0.0)
        return jax.lax.dot_general(ind, ones, (((1,), (0,)), ((), ())),
                                   preferred_element_type=jnp.float32)[:, 0]

    cm0 = jnp.max(x[:, 0:64], axis=1)
    lo_f, hi_f = cm0, cm0
    for c in range(1, 32):
        mc = jnp.max(x[:, c * 64:(c + 1) * 64], axis=1)
        lo_f = jnp.minimum(lo_f, mc)
        hi_f = jnp.maximum(hi_f, mc)
    lo_u = _f2u(lo_f)
    hi_u = _f2u(hi_f) + 1
    c_lo = cnt_ge(lo_f)
    c_hi = jnp.zeros_like(c_lo)
    act = jnp.where((c_lo == _KF) | ((hi_u - lo_u) <= 1), 0.0, 1.0)

    def w_cond(st):
        return jnp.max(st[5]) > 0.0

    def w_body(st):
        it, lo_u, hi_u, c_lo, c_hi, act = st
        lof, hif = _u2f(lo_u), _u2f(hi_u)
        frac = (c_lo - _KF) / jnp.maximum(c_lo - c_hi, 1.0)
        p_int = _f2u(lof + (hif - lof) * frac)
        p_mid = lo_u + jax.lax.shift_right_logical(hi_u - lo_u, 1)
        piv = jnp.where((it & 1) == 1, p_mid, p_int)
        piv = jnp.clip(piv, lo_u + 1, hi_u - 1)
        c = cnt_ge(_u2f(piv))
        ge = c >= _KF
        active = act > 0.0
        upd_lo = active & ge
        upd_hi = active & jnp.logical_not(ge)
        lo_u = jnp.where(upd_lo, piv, lo_u)
        c_lo = jnp.where(upd_lo, c, c_lo)
        hi_u = jnp.where(upd_hi, piv, hi_u)
        c_hi = jnp.where(upd_hi, c, c_hi)
        act = jnp.where((c == _KF) | ((hi_u - lo_u) <= 1), 0.0, act)
        return (it + jnp.int32(1), lo_u, hi_u, c_lo, c_hi, act)

    st = (jnp.int32(0), lo_u, hi_u, c_lo, c_hi, act)
    _, lo_u, _, _, _, _ = jax.lax.while_loop(w_cond, w_body, st)

    lof = _u2f(lo_u)
    tval = jnp.min(jnp.where(x >= lof[:, None], x, jnp.float32(jnp.inf)),
                   axis=1)
    gt = x > tval[:, None]
    cnt_gt = jax.lax.dot_general(jnp.where(gt, 1.0, 0.0), ones,
                                 (((1,), (0,)), ((), ())),
                                 preferred_element_type=jnp.float32)[:, 0]
    sum_gt = jnp.sum(jnp.where(gt, x, 0.0), axis=1)
    topsum = sum_gt + (_KF - cnt_gt) * tval
    eta = topsum / _K - row_sum / _LD

    e = E_ref[...]
    h = jax.lax.dot_general(e, W1_ref[...], (((1,), (0,)), ((), ())),
                            preferred_element_type=jnp.float32)
    h = jnp.maximum(h + b1_ref[...], 0.0)
    logit = jnp.sum(_bf16_round(h) * _bf16_round(W2_ref[...]), axis=1) + b2_ref[0]
    mu = jnp.maximum(logit, 0.0) + jnp.log1p(jnp.exp(-jnp.abs(logit)))

    vol_ref[...] = jnp.abs(eta) * mu

    @pl.when(i == 0)
    def _():
        musum_ref[0] = 0.0

    musum_ref[0] += jnp.sum(mu)


def _dream_body(scale_ref, vol_ref, M_ref, dream_ref):
    v = vol_ref[...]
    m = M_ref[...]
    dream_ref[...] = jnp.where(v[:, None] > _VOL_T, m * scale_ref[0], 0.0)


def kernel(manifold_state, S_matrix, state_energy, W1, b1, W2, b2, running_mu):
    vol, musum = pl.pallas_call(
        _vol_body,
        grid=(_NBLK,),
        in_specs=[
            pl.BlockSpec((_BLK, _LD), lambda i: (i, 0)),
            pl.BlockSpec((_BLK, _ED), lambda i: (i, 0)),
            pl.BlockSpec((_ED, _HID), lambda i: (0, 0)),
            pl.BlockSpec((1, _HID), lambda i: (0, 0)),
            pl.BlockSpec((1, _ED), lambda i: (0, 0)),
            pl.BlockSpec(memory_space=pltpu.SMEM),
        ],
        out_specs=[
            pl.BlockSpec((_BLK,), lambda i: (i,)),
            pl.BlockSpec(memory_space=pltpu.SMEM),
        ],
        out_shape=[
            jax.ShapeDtypeStruct((_B,), jnp.float32),
            jax.ShapeDtypeStruct((1,), jnp.float32),
        ],
    )(S_matrix, state_energy.astype(jnp.bfloat16), W1.astype(jnp.bfloat16),
      b1.reshape(1, _HID), W2.reshape(1, _ED), b2)

    mu_mean = musum[0] / _B
    new_running_mu = 0.9 * running_mu + 0.1 * mu_mean
    dynamic_delta = _DELTA_BASE * (1.0 + new_running_mu)
    scale = (1.0 + dynamic_delta).reshape(1)

    dream = pl.pallas_call(
        _dream_body,
        grid=(_NBLK2,),
        in_specs=[
            pl.BlockSpec(memory_space=pltpu.SMEM),
            pl.BlockSpec((_BLK2,), lambda i: (i,)),
            pl.BlockSpec((_BLK2, _LD), lambda i: (i, 0)),
        ],
        out_specs=pl.BlockSpec((_BLK2, _LD), lambda i: (i, 0)),
        out_shape=jax.ShapeDtypeStruct((_B, _LD), jnp.float32),
    )(scale, vol, manifold_state)

    return (vol, new_running_mu.reshape(()), dynamic_delta.reshape(()), dream)


# speedup vs baseline: 1.7170x; 1.7170x over previous
import jax
import jax.numpy as jnp
from jax.experimental import pallas as pl
from jax.experimental.pallas import tpu as pltpu

_B = 16384
_LD = 2048
_ED = 1024
_HID = 1024
_K = 32
_BLK = 256
_NBLK = _B // _BLK
_BLK2 = 512
_NBLK2 = _B // _BLK2
_DELTA_BASE = 0.01
_VOL_T = 0.5


def _bf16_round(x):
    u = jax.lax.bitcast_convert_type(x, jnp.int32)
    u = u + jnp.int32(0x7FFF) + ((u >> 16) & jnp.int32(1))
    u = u & jnp.int32(-65536)
    return jax.lax.bitcast_convert_type(u, jnp.float32)


def _vol_body(S_ref, E_ref, W1_ref, b1_ref, W2_ref, b2_ref,
              vol_ref, musum_ref):
    i = pl.program_id(0)

    x = S_ref[...]
    z = jax.lax.bitcast_convert_type(x, jnp.int32)
    z = z ^ ((z >> 31) & jnp.int32(0x7FFFFFFF))
    row_sum = jnp.sum(x, axis=1)

    cnt_nonneg = jnp.sum((z >= 0).astype(jnp.int32), axis=1)
    p = jnp.where(cnt_nonneg >= _K, jnp.int32(0), jnp.int32(-(2 ** 31)))
    for b in range(30, -1, -1):
        cand = p + jnp.int32(1 << b)
        cnt = jnp.sum((z >= cand[:, None]).astype(jnp.int32), axis=1)
        p = jnp.where(cnt >= _K, cand, p)

    gt = z > p[:, None]
    cnt_gt = jnp.sum(gt.astype(jnp.int32), axis=1)
    sum_gt = jnp.sum(jnp.where(gt, x, 0.0), axis=1)
    tz = p ^ ((p >> 31) & jnp.int32(0x7FFFFFFF))
    tval = jax.lax.bitcast_convert_type(tz, jnp.float32)
    topsum = sum_gt + (_K - cnt_gt).astype(jnp.float32) * tval
    eta = topsum / _K - row_sum / _LD

    e = E_ref[...]
    h = jax.lax.dot_general(e, W1_ref[...], (((1,), (0,)), ((), ())),
                            preferred_element_type=jnp.float32)
    h = jnp.maximum(h + b1_ref[...], 0.0)
    logit = jnp.sum(_bf16_round(h) * _bf16_round(W2_ref[...]), axis=1) \
        + b2_ref[0]
    mu = jnp.maximum(logit, 0.0) + jnp.log1p(jnp.exp(-jnp.abs(logit)))

    vol_ref[...] = jnp.abs(eta) * mu

    @pl.when(i == 0)
    def _():
        musum_ref[0] = 0.0

    musum_ref[0] += jnp.sum(mu)


def _dream_body(scale_ref, vol_ref, M_ref, dream_ref):
    v = vol_ref[...]
    m = M_ref[...]
    dream_ref[...] = jnp.where(v[:, None] > _VOL_T, m * scale_ref[0], 0.0)


def kernel(manifold_state, S_matrix, state_energy, W1, b1, W2, b2, running_mu):
    vol, musum = pl.pallas_call(
        _vol_body,
        grid=(_NBLK,),
        in_specs=[
            pl.BlockSpec((_BLK, _LD), lambda i: (i, 0)),
            pl.BlockSpec((_BLK, _ED), lambda i: (i, 0)),
            pl.BlockSpec((_ED, _HID), lambda i: (0, 0)),
            pl.BlockSpec((1, _HID), lambda i: (0, 0)),
            pl.BlockSpec((1, _ED), lambda i: (0, 0)),
            pl.BlockSpec(memory_space=pltpu.SMEM),
        ],
        out_specs=[
            pl.BlockSpec((_BLK,), lambda i: (i,)),
            pl.BlockSpec(memory_space=pltpu.SMEM),
        ],
        out_shape=[
            jax.ShapeDtypeStruct((_B,), jnp.float32),
            jax.ShapeDtypeStruct((1,), jnp.float32),
        ],
    )(S_matrix, state_energy.astype(jnp.bfloat16), W1.astype(jnp.bfloat16),
      b1.reshape(1, _HID), W2.reshape(1, _ED), b2)

    mu_mean = musum[0] / _B
    new_running_mu = 0.9 * running_mu + 0.1 * mu_mean
    dynamic_delta = _DELTA_BASE * (1.0 + new_running_mu)
    scale = (1.0 + dynamic_delta).reshape(1)

    dream = pl.pallas_call(
        _dream_body,
        grid=(_NBLK2,),
        in_specs=[
            pl.BlockSpec(memory_space=pltpu.SMEM),
            pl.BlockSpec((_BLK2,), lambda i: (i,)),
            pl.BlockSpec((_BLK2, _LD), lambda i: (i, 0)),
        ],
        out_specs=pl.BlockSpec((_BLK2, _LD), lambda i: (i, 0)),
        out_shape=jax.ShapeDtypeStruct((_B, _LD), jnp.float32),
    )(scale, vol, manifold_state)

    return (vol, new_running_mu.reshape(()), dynamic_delta.reshape(()), dream)
